# baseline (speedup 1.0000x reference)
import functools

import jax
import jax.numpy as jnp
from jax import lax
from jax.experimental import pallas as pl
from jax.experimental.pallas import tpu as pltpu
from jax.experimental.pallas import tpu_sc as plsc

H = 128
NPAD = 10240
NB = 512
EB = 256
NW = 32
NPH = 2
F32 = jnp.float32


def _pack_bf16_pair(p):
    lo = jax.lax.bitcast_convert_type(
        p[:, :H].astype(jnp.bfloat16), jnp.int16).astype(jnp.int32) & 0xFFFF
    hi = jax.lax.bitcast_convert_type(
        p[:, H:].astype(jnp.bfloat16), jnp.int16).astype(jnp.int32) << 16
    return lo | hi


def _unpack_bf16_pair(g):
    lo = jax.lax.bitcast_convert_type(g << 16, F32)
    hi = jax.lax.bitcast_convert_type(g & jnp.int32(-65536), F32)
    return jnp.concatenate([lo, hi], axis=1)


def _node_body(x_ref, w1_ref, b1_ref, w2_ref, b2_ref, ws_ref, wr_ref,
               nodebuf_ref, ps_ref, pr_ref, de0_ref, de1_ref, ds0_ref,
               ds1_ref):
    x = x_ref[...]
    h = jnp.dot(x, w1_ref[...], preferred_element_type=F32) + b1_ref[...]
    hs = h * jax.nn.sigmoid(h)
    o = jnp.dot(hs, w2_ref[...], preferred_element_type=F32) + b2_ref[...]
    dE0, dE1 = o[:, 0:1], o[:, 1:2]
    dS0, dS1 = o[:, 2:3], o[:, 3:4]
    l, m0, m1, m2 = o[:, 4:5], o[:, 5:6], o[:, 6:7], o[:, 7:8]
    a = m0 * m0
    b = m0 * m1
    c = m1 * m1 + m2 * m2
    nt0 = -l * dE1 + a * dS0 + b * dS1
    nt1 = l * dE0 + b * dS0 + c * dS1
    ge0 = a * dE0 + b * dE1
    ge1 = b * dE0 + c * dE1
    gs0 = -l * dS1
    gs1 = l * dS0
    z10 = jnp.zeros((nt0.shape[0], 10), F32)
    nodebuf_ref[...] = jnp.concatenate(
        [nt0, nt1, ge0, ge1, gs0, gs1, z10], axis=1)
    ps_ref[...] = _pack_bf16_pair(
        jnp.dot(x, ws_ref[...], preferred_element_type=F32))
    pr_ref[...] = _pack_bf16_pair(
        jnp.dot(x, wr_ref[...], preferred_element_type=F32))
    de0_ref[...] = o[:, 0]
    de1_ref[...] = o[:, 1]
    ds0_ref[...] = o[:, 2]
    ds1_ref[...] = o[:, 3]


def _node_call(xpad, w1, b1, w2, b2, ws, wr):
    n = xpad.shape[0]
    grid = (n // NB,)
    return pl.pallas_call(
        _node_body,
        grid=grid,
        in_specs=[
            pl.BlockSpec((NB, H), lambda i: (i, 0)),
            pl.BlockSpec((H, 4 * H), lambda i: (0, 0)),
            pl.BlockSpec((1, 4 * H), lambda i: (0, 0)),
            pl.BlockSpec((4 * H, 8), lambda i: (0, 0)),
            pl.BlockSpec((1, 8), lambda i: (0, 0)),
            pl.BlockSpec((H, 2 * H), lambda i: (0, 0)),
            pl.BlockSpec((H, 2 * H), lambda i: (0, 0)),
        ],
        out_specs=[
            pl.BlockSpec((NB, 16), lambda i: (i, 0)),
            pl.BlockSpec((NB, H), lambda i: (i, 0)),
            pl.BlockSpec((NB, H), lambda i: (i, 0)),
            pl.BlockSpec((NB,), lambda i: (i,)),
            pl.BlockSpec((NB,), lambda i: (i,)),
            pl.BlockSpec((NB,), lambda i: (i,)),
            pl.BlockSpec((NB,), lambda i: (i,)),
        ],
        out_shape=[
            jax.ShapeDtypeStruct((n, 16), F32),
            jax.ShapeDtypeStruct((n, H), jnp.int32),
            jax.ShapeDtypeStruct((n, H), jnp.int32),
            jax.ShapeDtypeStruct((n,), F32),
            jax.ShapeDtypeStruct((n,), F32),
            jax.ShapeDtypeStruct((n,), F32),
            jax.ShapeDtypeStruct((n,), F32),
        ],
    )(xpad, w1, b1, w2, b2, ws, wr)


def _sc_gather(ps, pr, s3d, r3d, e_total):
    nchunk = s3d.shape[1]
    gc = s3d.shape[2]
    epw = nchunk * gc
    mesh = plsc.VectorSubcoreMesh(core_axis_name="c", subcore_axis_name="s")

    @functools.partial(
        pl.kernel, mesh=mesh,
        out_type=[
            jax.ShapeDtypeStruct((e_total, H), jnp.int32),
            jax.ShapeDtypeStruct((e_total, H), jnp.int32),
        ],
        scratch_types=[
            pltpu.VMEM((nchunk, gc), jnp.int32),
            pltpu.VMEM((nchunk, gc), jnp.int32),
            pltpu.VMEM((gc, H), jnp.int32),
            pltpu.VMEM((gc, H), jnp.int32),
            pltpu.SemaphoreType.DMA,
        ],
    )
    def k(ps_hbm, pr_hbm, s_hbm, r_hbm, gs_out, gr_out,
          sidx_v, ridx_v, gs_v, gr_v, sem):
        wid = lax.axis_index("s") * 2 + lax.axis_index("c")
        base = wid * epw
        pltpu.sync_copy(s_hbm.at[wid], sidx_v)
        pltpu.sync_copy(r_hbm.at[wid], ridx_v)

        def chunk(j, carry):
            pltpu.async_copy(ps_hbm.at[sidx_v.at[j]], gs_v, sem).wait()
            pltpu.async_copy(pr_hbm.at[ridx_v.at[j]], gr_v, sem).wait()
            off = base + j * gc
            pltpu.sync_copy(gs_v, gs_out.at[pl.ds(off, gc)])
            pltpu.sync_copy(gr_v, gr_out.at[pl.ds(off, gc)])
            return carry

        lax.fori_loop(0, nchunk, chunk, 0)

    return k(ps, pr, s3d, r3d)


def _edge_body(ea_ref, gs_ref, gr_ref, a_ref, b1_ref, w2_ref, b2_ref,
               l_ref, a_ref_o, b_ref_o, c_ref_o):
    z = (jnp.dot(ea_ref[...].astype(jnp.bfloat16), a_ref[...],
                 preferred_element_type=F32)
         + _unpack_bf16_pair(gs_ref[...]) + _unpack_bf16_pair(gr_ref[...])
         + b1_ref[...])
    hs = z * jax.nn.sigmoid(z)
    lm_t = jax.lax.dot_general(
        w2_ref[...], hs.astype(jnp.bfloat16),
        (((0,), (1,)), ((), ())), preferred_element_type=F32) + b2_ref[...]
    l = lm_t[0, :]
    m0 = lm_t[1, :]
    m1 = lm_t[2, :]
    m2 = lm_t[3, :]
    l_ref[...] = l
    a_ref_o[...] = m0 * m0
    b_ref_o[...] = m0 * m1
    c_ref_o[...] = m1 * m1 + m2 * m2


def _edge_call(edge_attr, gs, gr, a_e, b1e, w2e, b2e, off):
    e = gs.shape[0]
    grid = (e // EB,)
    return pl.pallas_call(
        _edge_body,
        grid=grid,
        in_specs=[
            pl.BlockSpec((EB, H), lambda i, off=off: (i + off, 0)),
            pl.BlockSpec((EB, H), lambda i: (i, 0)),
            pl.BlockSpec((EB, H), lambda i: (i, 0)),
            pl.BlockSpec((H, 2 * H), lambda i: (0, 0)),
            pl.BlockSpec((1, 2 * H), lambda i: (0, 0)),
            pl.BlockSpec((2 * H, 4), lambda i: (0, 0)),
            pl.BlockSpec((4, 1), lambda i: (0, 0)),
        ],
        out_specs=[
            pl.BlockSpec((EB,), lambda i: (i,)),
            pl.BlockSpec((EB,), lambda i: (i,)),
            pl.BlockSpec((EB,), lambda i: (i,)),
            pl.BlockSpec((EB,), lambda i: (i,)),
        ],
        out_shape=[
            jax.ShapeDtypeStruct((e,), F32),
            jax.ShapeDtypeStruct((e,), F32),
            jax.ShapeDtypeStruct((e,), F32),
            jax.ShapeDtypeStruct((e,), F32),
        ],
    )(edge_attr, gs, gr, a_e, b1e, w2e, b2e)


def _sc_scatter(l_arr, a_arr, b_arr, c_arr, s2d, r2d, de0, de1, ds0, ds1):
    e_total = l_arr.shape[0]
    epw = e_total // NW
    npd = de0.shape[0]
    mesh = plsc.VectorSubcoreMesh(core_axis_name="c", subcore_axis_name="s")

    @functools.partial(
        pl.kernel, mesh=mesh,
        out_type=jax.ShapeDtypeStruct((NW, 2 * NPAD), F32),
        compiler_params=pltpu.CompilerParams(needs_layout_passes=False),
        scratch_types=[
            pltpu.VMEM((epw,), F32),
            pltpu.VMEM((epw,), F32),
            pltpu.VMEM((epw,), F32),
            pltpu.VMEM((epw,), F32),
            pltpu.VMEM((epw,), jnp.int32),
            pltpu.VMEM((epw,), jnp.int32),
            pltpu.VMEM((npd,), F32),
            pltpu.VMEM((npd,), F32),
            pltpu.VMEM((npd,), F32),
            pltpu.VMEM((npd,), F32),
            pltpu.VMEM((2 * NPAD,), F32),
        ],
    )
    def k(l_hbm, a_hbm, b_hbm, c_hbm, s_hbm, r_hbm,
          de0_hbm, de1_hbm, ds0_hbm, ds1_hbm, out_hbm,
          l_v, a_v, b_v, c_v, sidx_v, ridx_v,
          de0_v, de1_v, ds0_v, ds1_v, acc_v):
        wid = lax.axis_index("s") * 2 + lax.axis_index("c")
        base = wid * epw
        pltpu.sync_copy(l_hbm.at[pl.ds(base, epw)], l_v)
        pltpu.sync_copy(a_hbm.at[pl.ds(base, epw)], a_v)
        pltpu.sync_copy(b_hbm.at[pl.ds(base, epw)], b_v)
        pltpu.sync_copy(c_hbm.at[pl.ds(base, epw)], c_v)
        pltpu.sync_copy(s_hbm.at[wid], sidx_v)
        pltpu.sync_copy(r_hbm.at[wid], ridx_v)
        pltpu.sync_copy(de0_hbm, de0_v)
        pltpu.sync_copy(de1_hbm, de1_v)
        pltpu.sync_copy(ds0_hbm, ds0_v)
        pltpu.sync_copy(ds1_hbm, ds1_v)

        def zero(i, carry):
            acc_v[pl.ds(i * 16, 16)] = jnp.zeros((16,), F32)
            return carry

        lax.fori_loop(0, (2 * NPAD) // 16, zero, 0)

        def step(i, carry):
            sl = pl.ds(i * 16, 16)
            sidx = sidx_v[sl]
            de0s = plsc.load_gather(de0_v, [sidx])
            de1s = plsc.load_gather(de1_v, [sidx])
            ds0s = plsc.load_gather(ds0_v, [sidx])
            ds1s = plsc.load_gather(ds1_v, [sidx])
            l = l_v[sl]
            a = a_v[sl]
            b = b_v[sl]
            c = c_v[sl]
            et0 = -l * de1s + a * ds0s + b * ds1s
            et1 = l * de0s + b * ds0s + c * ds1s
            ridx = ridx_v[sl]
            plsc.addupdate_scatter(acc_v, [ridx], et0)
            plsc.addupdate_scatter(acc_v, [ridx + NPAD], et1)
            return carry

        lax.fori_loop(0, epw // 16, step, 0)
        pltpu.sync_copy(acc_v, out_hbm.at[wid])

    return k(l_arr, a_arr, b_arr, c_arr, s2d, r2d, de0, de1, ds0, ds1)


def _combine_body(nb_ref, p0_ref, p1_ref, out_ref):
    s0 = jnp.sum(p0_ref[...], axis=0)
    s1 = jnp.sum(p1_ref[...], axis=0)
    nb = nb_ref[...]
    out0 = nb[:, 0] - s0
    out1 = nb[:, 1] - s1
    z10 = jnp.zeros((nb.shape[0], 10), F32)
    out_ref[...] = jnp.concatenate(
        [out0[:, None], out1[:, None], nb[:, 2:6], z10], axis=1)


def _combine_call(nodebuf, partials):
    n = nodebuf.shape[0]
    grid = (n // NB,)
    nblk = n // NB
    return pl.pallas_call(
        _combine_body,
        grid=grid,
        in_specs=[
            pl.BlockSpec((NB, 16), lambda i: (i, 0)),
            pl.BlockSpec((NW, NB), lambda i: (0, i)),
            pl.BlockSpec((NW, NB), lambda i, nblk=nblk: (0, i + nblk)),
        ],
        out_specs=pl.BlockSpec((NB, 16), lambda i: (i, 0)),
        out_shape=jax.ShapeDtypeStruct((n, 16), F32),
    )(nodebuf, partials, partials)


def kernel(node_attr, edge_attr, params, edge_index):
    n = node_attr.shape[0]
    e = edge_attr.shape[0]
    senders = edge_index[0]
    receivers = edge_index[1]
    p = params

    w1n = jnp.concatenate([p["energy"]["W1"], p["entropy"]["W1"],
                           p["L_n"]["W1"], p["M_n"]["W1"]], axis=1)
    b1n = jnp.concatenate([p["energy"]["b1"], p["entropy"]["b1"],
                           p["L_n"]["b1"], p["M_n"]["b1"]])[None, :]
    w2n = jnp.zeros((4 * H, 8), F32)
    w2n = w2n.at[0:H, 0:2].set(p["energy"]["W2"])
    w2n = w2n.at[H:2 * H, 2:4].set(p["entropy"]["W2"])
    w2n = w2n.at[2 * H:3 * H, 4:5].set(p["L_n"]["W2"])
    w2n = w2n.at[3 * H:4 * H, 5:8].set(p["M_n"]["W2"])
    b2n = jnp.concatenate([p["energy"]["b2"], p["entropy"]["b2"],
                           p["L_n"]["b2"], p["M_n"]["b2"]])[None, :]

    a_e = jnp.concatenate([p["L_e"]["W1"][:H], p["M_e"]["W1"][:H]], axis=1)
    ws = jnp.concatenate([p["L_e"]["W1"][H:2 * H],
                          p["M_e"]["W1"][H:2 * H]], axis=1)
    wr = jnp.concatenate([p["L_e"]["W1"][2 * H:3 * H],
                          p["M_e"]["W1"][2 * H:3 * H]], axis=1)
    b1e = jnp.concatenate([p["L_e"]["b1"], p["M_e"]["b1"]])[None, :]
    w2e = jnp.zeros((2 * H, 4), F32)
    w2e = w2e.at[0:H, 0:1].set(p["L_e"]["W2"])
    w2e = w2e.at[H:2 * H, 1:4].set(p["M_e"]["W2"])
    b2e = jnp.concatenate([p["L_e"]["b2"], p["M_e"]["b2"]])[:, None]

    xpad = jnp.zeros((NPAD, H), F32).at[:n].set(node_attr)
    nodebuf, ps, pr, de0, de1, ds0, ds1 = _node_call(
        xpad, w1n, b1n, w2n, b2n, ws, wr)

    eh = e // NPH
    epw_p = eh // NW
    gcp = 40
    nchunk_p = epw_p // gcp
    a_bf = a_e.astype(jnp.bfloat16)
    w2_bf = w2e.astype(jnp.bfloat16)
    parts = []
    for p in range(NPH):
        sl = slice(p * eh, (p + 1) * eh)
        s3d = senders[sl].reshape(NW, nchunk_p, gcp)
        r3d = receivers[sl].reshape(NW, nchunk_p, gcp)
        gs_i, gr_i = _sc_gather(ps, pr, s3d, r3d, eh)
        parts.append(_edge_call(edge_attr, gs_i, gr_i, a_bf, b1e, w2_bf,
                                b2e, p * (eh // EB)))

    l_arr, a_arr, b_arr, c_arr = (
        jnp.concatenate([parts[p][j] for p in range(NPH)]) for j in range(4))

    epw = e // NW
    partials = _sc_scatter(l_arr, a_arr, b_arr, c_arr,
                           senders.reshape(NW, epw),
                           receivers.reshape(NW, epw),
                           de0, de1, ds0, ds1)

    res = _combine_call(nodebuf, partials)
    dzdt = res[:n, 0:2]
    deg_e = res[:n, 2:4].reshape(n, 2, 1)
    deg_s = res[:n, 4:6].reshape(n, 2, 1)
    return (dzdt, deg_e, deg_s)


# speedup vs baseline: 1.6442x; 1.6442x over previous
import functools

import jax
import jax.numpy as jnp
from jax import lax
from jax.experimental import pallas as pl
from jax.experimental.pallas import tpu as pltpu
from jax.experimental.pallas import tpu_sc as plsc

H = 128
NPAD = 10240
NB = 512
EB = 512
NW = 32
NPH = 5
F32 = jnp.float32


def _pack_bf16_pair(p):
    lo = jax.lax.bitcast_convert_type(
        p[:, :H].astype(jnp.bfloat16), jnp.int16).astype(jnp.int32) & 0xFFFF
    hi = jax.lax.bitcast_convert_type(
        p[:, H:].astype(jnp.bfloat16), jnp.int16).astype(jnp.int32) << 16
    return lo | hi


def _unpack_bf16_pair(g):
    lo = jax.lax.bitcast_convert_type(g << 16, F32)
    hi = jax.lax.bitcast_convert_type(g & jnp.int32(-65536), F32)
    return jnp.concatenate([lo, hi], axis=1)


def _node_body(x_ref, w1_ref, b1_ref, w2_ref, b2_ref, ws_ref, wr_ref,
               nodebuf_ref, ps_ref, pr_ref, de0_ref, de1_ref, ds0_ref,
               ds1_ref):
    x = x_ref[...]
    h = jnp.dot(x, w1_ref[...], preferred_element_type=F32) + b1_ref[...]
    hs = h * jax.nn.sigmoid(h)
    o = jnp.dot(hs, w2_ref[...], preferred_element_type=F32) + b2_ref[...]
    dE0, dE1 = o[:, 0:1], o[:, 1:2]
    dS0, dS1 = o[:, 2:3], o[:, 3:4]
    l, m0, m1, m2 = o[:, 4:5], o[:, 5:6], o[:, 6:7], o[:, 7:8]
    a = m0 * m0
    b = m0 * m1
    c = m1 * m1 + m2 * m2
    nt0 = -l * dE1 + a * dS0 + b * dS1
    nt1 = l * dE0 + b * dS0 + c * dS1
    ge0 = a * dE0 + b * dE1
    ge1 = b * dE0 + c * dE1
    gs0 = -l * dS1
    gs1 = l * dS0
    z10 = jnp.zeros((nt0.shape[0], 10), F32)
    nodebuf_ref[...] = jnp.concatenate(
        [nt0, nt1, ge0, ge1, gs0, gs1, z10], axis=1)
    ps_ref[...] = _pack_bf16_pair(
        jnp.dot(x, ws_ref[...], preferred_element_type=F32))
    pr_ref[...] = _pack_bf16_pair(
        jnp.dot(x, wr_ref[...], preferred_element_type=F32))
    de0_ref[...] = o[:, 0]
    de1_ref[...] = o[:, 1]
    ds0_ref[...] = o[:, 2]
    ds1_ref[...] = o[:, 3]


def _node_call(xpad, w1, b1, w2, b2, ws, wr):
    n = xpad.shape[0]
    grid = (n // NB,)
    return pl.pallas_call(
        _node_body,
        grid=grid,
        in_specs=[
            pl.BlockSpec((NB, H), lambda i: (i, 0)),
            pl.BlockSpec((H, 4 * H), lambda i: (0, 0)),
            pl.BlockSpec((1, 4 * H), lambda i: (0, 0)),
            pl.BlockSpec((4 * H, 8), lambda i: (0, 0)),
            pl.BlockSpec((1, 8), lambda i: (0, 0)),
            pl.BlockSpec((H, 2 * H), lambda i: (0, 0)),
            pl.BlockSpec((H, 2 * H), lambda i: (0, 0)),
        ],
        out_specs=[
            pl.BlockSpec((NB, 16), lambda i: (i, 0)),
            pl.BlockSpec((NB, H), lambda i: (i, 0)),
            pl.BlockSpec((NB, H), lambda i: (i, 0)),
            pl.BlockSpec((NB,), lambda i: (i,)),
            pl.BlockSpec((NB,), lambda i: (i,)),
            pl.BlockSpec((NB,), lambda i: (i,)),
            pl.BlockSpec((NB,), lambda i: (i,)),
        ],
        out_shape=[
            jax.ShapeDtypeStruct((n, 16), F32),
            jax.ShapeDtypeStruct((n, H), jnp.int32),
            jax.ShapeDtypeStruct((n, H), jnp.int32),
            jax.ShapeDtypeStruct((n,), F32),
            jax.ShapeDtypeStruct((n,), F32),
            jax.ShapeDtypeStruct((n,), F32),
            jax.ShapeDtypeStruct((n,), F32),
        ],
    )(xpad, w1, b1, w2, b2, ws, wr)


def _sc_gather(ps, pr, s3d, r3d, e_total):
    nchunk = s3d.shape[1]
    gc = s3d.shape[2]
    epw = nchunk * gc
    mesh = plsc.VectorSubcoreMesh(core_axis_name="c", subcore_axis_name="s")

    @functools.partial(
        pl.kernel, mesh=mesh,
        out_type=[
            jax.ShapeDtypeStruct((e_total, H), jnp.int32),
            jax.ShapeDtypeStruct((e_total, H), jnp.int32),
        ],
        scratch_types=[
            pltpu.VMEM((nchunk, gc), jnp.int32),
            pltpu.VMEM((nchunk, gc), jnp.int32),
            pltpu.VMEM((gc, H), jnp.int32),
            pltpu.VMEM((gc, H), jnp.int32),
            pltpu.SemaphoreType.DMA,
        ],
    )
    def k(ps_hbm, pr_hbm, s_hbm, r_hbm, gs_out, gr_out,
          sidx_v, ridx_v, gs_v, gr_v, sem):
        wid = lax.axis_index("s") * 2 + lax.axis_index("c")
        base = wid * epw
        pltpu.sync_copy(s_hbm.at[wid], sidx_v)
        pltpu.sync_copy(r_hbm.at[wid], ridx_v)

        def chunk(j, carry):
            pltpu.async_copy(ps_hbm.at[sidx_v.at[j]], gs_v, sem).wait()
            pltpu.async_copy(pr_hbm.at[ridx_v.at[j]], gr_v, sem).wait()
            off = base + j * gc
            pltpu.sync_copy(gs_v, gs_out.at[pl.ds(off, gc)])
            pltpu.sync_copy(gr_v, gr_out.at[pl.ds(off, gc)])
            return carry

        lax.fori_loop(0, nchunk, chunk, 0)

    return k(ps, pr, s3d, r3d)


def _edge_body(ea_ref, gs_ref, gr_ref, a_ref, b1_ref, w2_ref, b2_ref,
               l_ref, a_ref_o, b_ref_o, c_ref_o):
    z = (jnp.dot(ea_ref[...].astype(jnp.bfloat16), a_ref[...],
                 preferred_element_type=F32)
         + _unpack_bf16_pair(gs_ref[...]) + _unpack_bf16_pair(gr_ref[...])
         + b1_ref[...])
    hs = z * jax.nn.sigmoid(z)
    lm_t = jax.lax.dot_general(
        w2_ref[...], hs.astype(jnp.bfloat16),
        (((0,), (1,)), ((), ())), preferred_element_type=F32) + b2_ref[...]
    l = lm_t[0, :]
    m0 = lm_t[1, :]
    m1 = lm_t[2, :]
    m2 = lm_t[3, :]
    l_ref[...] = l
    a_ref_o[...] = m0 * m0
    b_ref_o[...] = m0 * m1
    c_ref_o[...] = m1 * m1 + m2 * m2


def _edge_call(edge_attr, gs, gr, a_e, b1e, w2e, b2e, off):
    e = gs.shape[0]
    grid = (e // EB,)
    return pl.pallas_call(
        _edge_body,
        grid=grid,
        in_specs=[
            pl.BlockSpec((EB, H), lambda i, off=off: (i + off, 0)),
            pl.BlockSpec((EB, H), lambda i: (i, 0)),
            pl.BlockSpec((EB, H), lambda i: (i, 0)),
            pl.BlockSpec((H, 2 * H), lambda i: (0, 0)),
            pl.BlockSpec((1, 2 * H), lambda i: (0, 0)),
            pl.BlockSpec((2 * H, 4), lambda i: (0, 0)),
            pl.BlockSpec((4, 1), lambda i: (0, 0)),
        ],
        out_specs=[
            pl.BlockSpec((EB,), lambda i: (i,)),
            pl.BlockSpec((EB,), lambda i: (i,)),
            pl.BlockSpec((EB,), lambda i: (i,)),
            pl.BlockSpec((EB,), lambda i: (i,)),
        ],
        out_shape=[
            jax.ShapeDtypeStruct((e,), F32),
            jax.ShapeDtypeStruct((e,), F32),
            jax.ShapeDtypeStruct((e,), F32),
            jax.ShapeDtypeStruct((e,), F32),
        ],
    )(edge_attr, gs, gr, a_e, b1e, w2e, b2e)


def _sc_scatter(l_arr, a_arr, b_arr, c_arr, s2d, r2d, de0, de1, ds0, ds1):
    e_total = l_arr.shape[0]
    epw = e_total // NW
    npd = de0.shape[0]
    mesh = plsc.VectorSubcoreMesh(core_axis_name="c", subcore_axis_name="s")

    @functools.partial(
        pl.kernel, mesh=mesh,
        out_type=jax.ShapeDtypeStruct((NW, 2 * NPAD), F32),
        compiler_params=pltpu.CompilerParams(needs_layout_passes=False),
        scratch_types=[
            pltpu.VMEM((epw,), F32),
            pltpu.VMEM((epw,), F32),
            pltpu.VMEM((epw,), F32),
            pltpu.VMEM((epw,), F32),
            pltpu.VMEM((epw,), jnp.int32),
            pltpu.VMEM((epw,), jnp.int32),
            pltpu.VMEM((npd,), F32),
            pltpu.VMEM((npd,), F32),
            pltpu.VMEM((npd,), F32),
            pltpu.VMEM((npd,), F32),
            pltpu.VMEM((2 * NPAD,), F32),
        ],
    )
    def k(l_hbm, a_hbm, b_hbm, c_hbm, s_hbm, r_hbm,
          de0_hbm, de1_hbm, ds0_hbm, ds1_hbm, out_hbm,
          l_v, a_v, b_v, c_v, sidx_v, ridx_v,
          de0_v, de1_v, ds0_v, ds1_v, acc_v):
        wid = lax.axis_index("s") * 2 + lax.axis_index("c")
        base = wid * epw
        pltpu.sync_copy(l_hbm.at[pl.ds(base, epw)], l_v)
        pltpu.sync_copy(a_hbm.at[pl.ds(base, epw)], a_v)
        pltpu.sync_copy(b_hbm.at[pl.ds(base, epw)], b_v)
        pltpu.sync_copy(c_hbm.at[pl.ds(base, epw)], c_v)
        pltpu.sync_copy(s_hbm.at[wid], sidx_v)
        pltpu.sync_copy(r_hbm.at[wid], ridx_v)
        pltpu.sync_copy(de0_hbm, de0_v)
        pltpu.sync_copy(de1_hbm, de1_v)
        pltpu.sync_copy(ds0_hbm, ds0_v)
        pltpu.sync_copy(ds1_hbm, ds1_v)

        def zero(i, carry):
            acc_v[pl.ds(i * 16, 16)] = jnp.zeros((16,), F32)
            return carry

        lax.fori_loop(0, (2 * NPAD) // 16, zero, 0)

        def step(i, carry):
            sl = pl.ds(i * 16, 16)
            sidx = sidx_v[sl]
            de0s = plsc.load_gather(de0_v, [sidx])
            de1s = plsc.load_gather(de1_v, [sidx])
            ds0s = plsc.load_gather(ds0_v, [sidx])
            ds1s = plsc.load_gather(ds1_v, [sidx])
            l = l_v[sl]
            a = a_v[sl]
            b = b_v[sl]
            c = c_v[sl]
            et0 = -l * de1s + a * ds0s + b * ds1s
            et1 = l * de0s + b * ds0s + c * ds1s
            ridx = ridx_v[sl]
            plsc.addupdate_scatter(acc_v, [ridx], et0)
            plsc.addupdate_scatter(acc_v, [ridx + NPAD], et1)
            return carry

        lax.fori_loop(0, epw // 16, step, 0)
        pltpu.sync_copy(acc_v, out_hbm.at[wid])

    return k(l_arr, a_arr, b_arr, c_arr, s2d, r2d, de0, de1, ds0, ds1)


def _combine_body(nb_ref, p0_ref, p1_ref, out_ref):
    s0 = jnp.sum(p0_ref[...], axis=0)
    s1 = jnp.sum(p1_ref[...], axis=0)
    nb = nb_ref[...]
    out0 = nb[:, 0] - s0
    out1 = nb[:, 1] - s1
    z10 = jnp.zeros((nb.shape[0], 10), F32)
    out_ref[...] = jnp.concatenate(
        [out0[:, None], out1[:, None], nb[:, 2:6], z10], axis=1)


def _combine_call(nodebuf, partials):
    n = nodebuf.shape[0]
    grid = (n // NB,)
    nblk = n // NB
    return pl.pallas_call(
        _combine_body,
        grid=grid,
        in_specs=[
            pl.BlockSpec((NB, 16), lambda i: (i, 0)),
            pl.BlockSpec((NW, NB), lambda i: (0, i)),
            pl.BlockSpec((NW, NB), lambda i, nblk=nblk: (0, i + nblk)),
        ],
        out_specs=pl.BlockSpec((NB, 16), lambda i: (i, 0)),
        out_shape=jax.ShapeDtypeStruct((n, 16), F32),
    )(nodebuf, partials, partials)


def kernel(node_attr, edge_attr, params, edge_index):
    n = node_attr.shape[0]
    e = edge_attr.shape[0]
    senders = edge_index[0]
    receivers = edge_index[1]
    p = params

    w1n = jnp.concatenate([p["energy"]["W1"], p["entropy"]["W1"],
                           p["L_n"]["W1"], p["M_n"]["W1"]], axis=1)
    b1n = jnp.concatenate([p["energy"]["b1"], p["entropy"]["b1"],
                           p["L_n"]["b1"], p["M_n"]["b1"]])[None, :]
    w2n = jnp.zeros((4 * H, 8), F32)
    w2n = w2n.at[0:H, 0:2].set(p["energy"]["W2"])
    w2n = w2n.at[H:2 * H, 2:4].set(p["entropy"]["W2"])
    w2n = w2n.at[2 * H:3 * H, 4:5].set(p["L_n"]["W2"])
    w2n = w2n.at[3 * H:4 * H, 5:8].set(p["M_n"]["W2"])
    b2n = jnp.concatenate([p["energy"]["b2"], p["entropy"]["b2"],
                           p["L_n"]["b2"], p["M_n"]["b2"]])[None, :]

    a_e = jnp.concatenate([p["L_e"]["W1"][:H], p["M_e"]["W1"][:H]], axis=1)
    ws = jnp.concatenate([p["L_e"]["W1"][H:2 * H],
                          p["M_e"]["W1"][H:2 * H]], axis=1)
    wr = jnp.concatenate([p["L_e"]["W1"][2 * H:3 * H],
                          p["M_e"]["W1"][2 * H:3 * H]], axis=1)
    b1e = jnp.concatenate([p["L_e"]["b1"], p["M_e"]["b1"]])[None, :]
    w2e = jnp.zeros((2 * H, 4), F32)
    w2e = w2e.at[0:H, 0:1].set(p["L_e"]["W2"])
    w2e = w2e.at[H:2 * H, 1:4].set(p["M_e"]["W2"])
    b2e = jnp.concatenate([p["L_e"]["b2"], p["M_e"]["b2"]])[:, None]

    xpad = jnp.zeros((NPAD, H), F32).at[:n].set(node_attr)
    nodebuf, ps, pr, de0, de1, ds0, ds1 = _node_call(
        xpad, w1n, b1n, w2n, b2n, ws, wr)

    eh = e // NPH
    epw_p = eh // NW
    gcp = 80
    nchunk_p = epw_p // gcp
    a_bf = a_e.astype(jnp.bfloat16)
    w2_bf = w2e.astype(jnp.bfloat16)
    parts = []
    for p in range(NPH):
        sl = slice(p * eh, (p + 1) * eh)
        s3d = senders[sl].reshape(NW, nchunk_p, gcp)
        r3d = receivers[sl].reshape(NW, nchunk_p, gcp)
        gs_i, gr_i = _sc_gather(ps, pr, s3d, r3d, eh)
        parts.append(_edge_call(edge_attr, gs_i, gr_i, a_bf, b1e, w2_bf,
                                b2e, p * (eh // EB)))

    l_arr, a_arr, b_arr, c_arr = (
        jnp.concatenate([parts[p][j] for p in range(NPH)]) for j in range(4))

    epw = e // NW
    partials = _sc_scatter(l_arr, a_arr, b_arr, c_arr,
                           senders.reshape(NW, epw),
                           receivers.reshape(NW, epw),
                           de0, de1, ds0, ds1)

    res = _combine_call(nodebuf, partials)
    dzdt = res[:n, 0:2]
    deg_e = res[:n, 2:4].reshape(n, 2, 1)
    deg_s = res[:n, 4:6].reshape(n, 2, 1)
    return (dzdt, deg_e, deg_s)


# speedup vs baseline: 1.9236x; 1.1700x over previous
import functools

import jax
import jax.numpy as jnp
from jax import lax
from jax.experimental import pallas as pl
from jax.experimental.pallas import tpu as pltpu
from jax.experimental.pallas import tpu_sc as plsc

H = 128
NPAD = 10240
NB = 512
EB = 1280
NW = 32
NPH = 5
F32 = jnp.float32


def _pack_bf16_pair(p):
    lo = jax.lax.bitcast_convert_type(
        p[:, :H].astype(jnp.bfloat16), jnp.int16).astype(jnp.int32) & 0xFFFF
    hi = jax.lax.bitcast_convert_type(
        p[:, H:].astype(jnp.bfloat16), jnp.int16).astype(jnp.int32) << 16
    return lo | hi


def _unpack_bf16_pair(g):
    lo = jax.lax.bitcast_convert_type(g << 16, F32)
    hi = jax.lax.bitcast_convert_type(g & jnp.int32(-65536), F32)
    return jnp.concatenate([lo, hi], axis=1)


def _node_body(x_ref, w1_ref, b1_ref, w2_ref, b2_ref, ws_ref, wr_ref,
               nodebuf_ref, ps_ref, pr_ref, de0_ref, de1_ref, ds0_ref,
               ds1_ref):
    x = x_ref[...]
    h = jnp.dot(x, w1_ref[...], preferred_element_type=F32) + b1_ref[...]
    hs = h * jax.nn.sigmoid(h)
    o = jnp.dot(hs, w2_ref[...], preferred_element_type=F32) + b2_ref[...]
    dE0, dE1 = o[:, 0:1], o[:, 1:2]
    dS0, dS1 = o[:, 2:3], o[:, 3:4]
    l, m0, m1, m2 = o[:, 4:5], o[:, 5:6], o[:, 6:7], o[:, 7:8]
    a = m0 * m0
    b = m0 * m1
    c = m1 * m1 + m2 * m2
    nt0 = -l * dE1 + a * dS0 + b * dS1
    nt1 = l * dE0 + b * dS0 + c * dS1
    ge0 = a * dE0 + b * dE1
    ge1 = b * dE0 + c * dE1
    gs0 = -l * dS1
    gs1 = l * dS0
    z10 = jnp.zeros((nt0.shape[0], 10), F32)
    nodebuf_ref[...] = jnp.concatenate(
        [nt0, nt1, ge0, ge1, gs0, gs1, z10], axis=1)
    ps_ref[...] = _pack_bf16_pair(
        jnp.dot(x, ws_ref[...], preferred_element_type=F32))
    pr_ref[...] = _pack_bf16_pair(
        jnp.dot(x, wr_ref[...], preferred_element_type=F32))
    de0_ref[...] = o[:, 0]
    de1_ref[...] = o[:, 1]
    ds0_ref[...] = o[:, 2]
    ds1_ref[...] = o[:, 3]


def _node_call(xpad, w1, b1, w2, b2, ws, wr):
    n = xpad.shape[0]
    grid = (n // NB,)
    return pl.pallas_call(
        _node_body,
        grid=grid,
        in_specs=[
            pl.BlockSpec((NB, H), lambda i: (i, 0)),
            pl.BlockSpec((H, 4 * H), lambda i: (0, 0)),
            pl.BlockSpec((1, 4 * H), lambda i: (0, 0)),
            pl.BlockSpec((4 * H, 8), lambda i: (0, 0)),
            pl.BlockSpec((1, 8), lambda i: (0, 0)),
            pl.BlockSpec((H, 2 * H), lambda i: (0, 0)),
            pl.BlockSpec((H, 2 * H), lambda i: (0, 0)),
        ],
        out_specs=[
            pl.BlockSpec((NB, 16), lambda i: (i, 0)),
            pl.BlockSpec((NB, H), lambda i: (i, 0)),
            pl.BlockSpec((NB, H), lambda i: (i, 0)),
            pl.BlockSpec((NB,), lambda i: (i,)),
            pl.BlockSpec((NB,), lambda i: (i,)),
            pl.BlockSpec((NB,), lambda i: (i,)),
            pl.BlockSpec((NB,), lambda i: (i,)),
        ],
        out_shape=[
            jax.ShapeDtypeStruct((n, 16), F32),
            jax.ShapeDtypeStruct((n, H), jnp.int32),
            jax.ShapeDtypeStruct((n, H), jnp.int32),
            jax.ShapeDtypeStruct((n,), F32),
            jax.ShapeDtypeStruct((n,), F32),
            jax.ShapeDtypeStruct((n,), F32),
            jax.ShapeDtypeStruct((n,), F32),
        ],
    )(xpad, w1, b1, w2, b2, ws, wr)


def _sc_gather(ps, pr, s3d, r3d, e_total):
    nchunk = s3d.shape[1]
    gc = s3d.shape[2]
    epw = nchunk * gc
    mesh = plsc.VectorSubcoreMesh(core_axis_name="c", subcore_axis_name="s")

    @functools.partial(
        pl.kernel, mesh=mesh,
        out_type=[
            jax.ShapeDtypeStruct((e_total, H), jnp.int32),
            jax.ShapeDtypeStruct((e_total, H), jnp.int32),
        ],
        scratch_types=[
            pltpu.VMEM((nchunk, gc), jnp.int32),
            pltpu.VMEM((nchunk, gc), jnp.int32),
            pltpu.VMEM((gc, H), jnp.int32),
            pltpu.VMEM((gc, H), jnp.int32),
            pltpu.SemaphoreType.DMA,
        ],
    )
    def k(ps_hbm, pr_hbm, s_hbm, r_hbm, gs_out, gr_out,
          sidx_v, ridx_v, gs_v, gr_v, sem):
        wid = lax.axis_index("s") * 2 + lax.axis_index("c")
        base = wid * epw
        pltpu.sync_copy(s_hbm.at[wid], sidx_v)
        pltpu.sync_copy(r_hbm.at[wid], ridx_v)

        def chunk(j, carry):
            pltpu.async_copy(ps_hbm.at[sidx_v.at[j]], gs_v, sem).wait()
            pltpu.async_copy(pr_hbm.at[ridx_v.at[j]], gr_v, sem).wait()
            off = base + j * gc
            pltpu.sync_copy(gs_v, gs_out.at[pl.ds(off, gc)])
            pltpu.sync_copy(gr_v, gr_out.at[pl.ds(off, gc)])
            return carry

        lax.fori_loop(0, nchunk, chunk, 0)

    return k(ps, pr, s3d, r3d)


def _edge_body(ea_ref, gs_ref, gr_ref, a_ref, b1_ref, w2_ref, b2_ref,
               lm4_ref):
    z = (jnp.dot(ea_ref[...].astype(jnp.bfloat16), a_ref[...],
                 preferred_element_type=F32)
         + _unpack_bf16_pair(gs_ref[...]) + _unpack_bf16_pair(gr_ref[...])
         + b1_ref[...])
    hs = z * jax.nn.sigmoid(z)
    lm_t = jax.lax.dot_general(
        w2_ref[...], hs.astype(jnp.bfloat16),
        (((0,), (1,)), ((), ())), preferred_element_type=F32) + b2_ref[...]
    l = lm_t[0:1, :]
    m0 = lm_t[1:2, :]
    m1 = lm_t[2:3, :]
    m2 = lm_t[3:4, :]
    lm4_ref[...] = jnp.concatenate(
        [l, m0 * m0, m0 * m1, m1 * m1 + m2 * m2], axis=0)


def _edge_call(edge_attr, gs, gr, a_e, b1e, w2e, b2e, off):
    e = gs.shape[0]
    grid = (e // EB,)
    return pl.pallas_call(
        _edge_body,
        grid=grid,
        in_specs=[
            pl.BlockSpec((EB, H), lambda i, off=off: (i + off, 0)),
            pl.BlockSpec((EB, H), lambda i: (i, 0)),
            pl.BlockSpec((EB, H), lambda i: (i, 0)),
            pl.BlockSpec((H, 2 * H), lambda i: (0, 0)),
            pl.BlockSpec((1, 2 * H), lambda i: (0, 0)),
            pl.BlockSpec((2 * H, 4), lambda i: (0, 0)),
            pl.BlockSpec((4, 1), lambda i: (0, 0)),
        ],
        out_specs=pl.BlockSpec((4, EB), lambda i: (0, i)),
        out_shape=jax.ShapeDtypeStruct((4, e), F32),
    )(edge_attr, gs, gr, a_e, b1e, w2e, b2e)


def _sc_scatter(l_arr, a_arr, b_arr, c_arr, s2d, r2d, de0, de1, ds0, ds1):
    e_total = l_arr.shape[0]
    epw = e_total // NW
    npd = de0.shape[0]
    mesh = plsc.VectorSubcoreMesh(core_axis_name="c", subcore_axis_name="s")

    @functools.partial(
        pl.kernel, mesh=mesh,
        out_type=jax.ShapeDtypeStruct((NW, 2 * NPAD), F32),
        compiler_params=pltpu.CompilerParams(needs_layout_passes=False),
        scratch_types=[
            pltpu.VMEM((epw,), F32),
            pltpu.VMEM((epw,), F32),
            pltpu.VMEM((epw,), F32),
            pltpu.VMEM((epw,), F32),
            pltpu.VMEM((epw,), jnp.int32),
            pltpu.VMEM((epw,), jnp.int32),
            pltpu.VMEM((npd,), F32),
            pltpu.VMEM((npd,), F32),
            pltpu.VMEM((npd,), F32),
            pltpu.VMEM((npd,), F32),
            pltpu.VMEM((2 * NPAD,), F32),
        ],
    )
    def k(l_hbm, a_hbm, b_hbm, c_hbm, s_hbm, r_hbm,
          de0_hbm, de1_hbm, ds0_hbm, ds1_hbm, out_hbm,
          l_v, a_v, b_v, c_v, sidx_v, ridx_v,
          de0_v, de1_v, ds0_v, ds1_v, acc_v):
        wid = lax.axis_index("s") * 2 + lax.axis_index("c")
        base = wid * epw
        pltpu.sync_copy(l_hbm.at[pl.ds(base, epw)], l_v)
        pltpu.sync_copy(a_hbm.at[pl.ds(base, epw)], a_v)
        pltpu.sync_copy(b_hbm.at[pl.ds(base, epw)], b_v)
        pltpu.sync_copy(c_hbm.at[pl.ds(base, epw)], c_v)
        pltpu.sync_copy(s_hbm.at[wid], sidx_v)
        pltpu.sync_copy(r_hbm.at[wid], ridx_v)
        pltpu.sync_copy(de0_hbm, de0_v)
        pltpu.sync_copy(de1_hbm, de1_v)
        pltpu.sync_copy(ds0_hbm, ds0_v)
        pltpu.sync_copy(ds1_hbm, ds1_v)

        def zero(i, carry):
            acc_v[pl.ds(i * 16, 16)] = jnp.zeros((16,), F32)
            return carry

        lax.fori_loop(0, (2 * NPAD) // 16, zero, 0)

        def step(i, carry):
            sl = pl.ds(i * 16, 16)
            sidx = sidx_v[sl]
            de0s = plsc.load_gather(de0_v, [sidx])
            de1s = plsc.load_gather(de1_v, [sidx])
            ds0s = plsc.load_gather(ds0_v, [sidx])
            ds1s = plsc.load_gather(ds1_v, [sidx])
            l = l_v[sl]
            a = a_v[sl]
            b = b_v[sl]
            c = c_v[sl]
            et0 = -l * de1s + a * ds0s + b * ds1s
            et1 = l * de0s + b * ds0s + c * ds1s
            ridx = ridx_v[sl]
            plsc.addupdate_scatter(acc_v, [ridx], et0)
            plsc.addupdate_scatter(acc_v, [ridx + NPAD], et1)
            return carry

        lax.fori_loop(0, epw // 16, step, 0)
        pltpu.sync_copy(acc_v, out_hbm.at[wid])

    return k(l_arr, a_arr, b_arr, c_arr, s2d, r2d, de0, de1, ds0, ds1)


def _combine_body(nb_ref, p0_ref, p1_ref, out_ref):
    s0 = jnp.sum(p0_ref[...], axis=0)
    s1 = jnp.sum(p1_ref[...], axis=0)
    nb = nb_ref[...]
    out0 = nb[:, 0] - s0
    out1 = nb[:, 1] - s1
    z10 = jnp.zeros((nb.shape[0], 10), F32)
    out_ref[...] = jnp.concatenate(
        [out0[:, None], out1[:, None], nb[:, 2:6], z10], axis=1)


def _combine_call(nodebuf, partials):
    n = nodebuf.shape[0]
    grid = (n // NB,)
    nblk = n // NB
    return pl.pallas_call(
        _combine_body,
        grid=grid,
        in_specs=[
            pl.BlockSpec((NB, 16), lambda i: (i, 0)),
            pl.BlockSpec((NW, NB), lambda i: (0, i)),
            pl.BlockSpec((NW, NB), lambda i, nblk=nblk: (0, i + nblk)),
        ],
        out_specs=pl.BlockSpec((NB, 16), lambda i: (i, 0)),
        out_shape=jax.ShapeDtypeStruct((n, 16), F32),
    )(nodebuf, partials, partials)


def kernel(node_attr, edge_attr, params, edge_index):
    n = node_attr.shape[0]
    e = edge_attr.shape[0]
    senders = edge_index[0]
    receivers = edge_index[1]
    p = params

    w1n = jnp.concatenate([p["energy"]["W1"], p["entropy"]["W1"],
                           p["L_n"]["W1"], p["M_n"]["W1"]], axis=1)
    b1n = jnp.concatenate([p["energy"]["b1"], p["entropy"]["b1"],
                           p["L_n"]["b1"], p["M_n"]["b1"]])[None, :]
    w2n = jnp.zeros((4 * H, 8), F32)
    w2n = w2n.at[0:H, 0:2].set(p["energy"]["W2"])
    w2n = w2n.at[H:2 * H, 2:4].set(p["entropy"]["W2"])
    w2n = w2n.at[2 * H:3 * H, 4:5].set(p["L_n"]["W2"])
    w2n = w2n.at[3 * H:4 * H, 5:8].set(p["M_n"]["W2"])
    b2n = jnp.concatenate([p["energy"]["b2"], p["entropy"]["b2"],
                           p["L_n"]["b2"], p["M_n"]["b2"]])[None, :]

    a_e = jnp.concatenate([p["L_e"]["W1"][:H], p["M_e"]["W1"][:H]], axis=1)
    ws = jnp.concatenate([p["L_e"]["W1"][H:2 * H],
                          p["M_e"]["W1"][H:2 * H]], axis=1)
    wr = jnp.concatenate([p["L_e"]["W1"][2 * H:3 * H],
                          p["M_e"]["W1"][2 * H:3 * H]], axis=1)
    b1e = jnp.concatenate([p["L_e"]["b1"], p["M_e"]["b1"]])[None, :]
    w2e = jnp.zeros((2 * H, 4), F32)
    w2e = w2e.at[0:H, 0:1].set(p["L_e"]["W2"])
    w2e = w2e.at[H:2 * H, 1:4].set(p["M_e"]["W2"])
    b2e = jnp.concatenate([p["L_e"]["b2"], p["M_e"]["b2"]])[:, None]

    xpad = jnp.zeros((NPAD, H), F32).at[:n].set(node_attr)
    nodebuf, ps, pr, de0, de1, ds0, ds1 = _node_call(
        xpad, w1n, b1n, w2n, b2n, ws, wr)

    eh = e // NPH
    epw_p = eh // NW
    gcp = 80
    nchunk_p = epw_p // gcp
    a_bf = a_e.astype(jnp.bfloat16)
    w2_bf = w2e.astype(jnp.bfloat16)
    parts = []
    for p in range(NPH):
        sl = slice(p * eh, (p + 1) * eh)
        s3d = senders[sl].reshape(NW, nchunk_p, gcp)
        r3d = receivers[sl].reshape(NW, nchunk_p, gcp)
        gs_i, gr_i = _sc_gather(ps, pr, s3d, r3d, eh)
        lm4 = _edge_call(edge_attr, gs_i, gr_i, a_bf, b1e, w2_bf,
                         b2e, p * (eh // EB))
        parts.append(tuple(lm4[j] for j in range(4)))

    l_arr, a_arr, b_arr, c_arr = (
        jnp.concatenate([parts[p][j] for p in range(NPH)]) for j in range(4))

    epw = e // NW
    partials = _sc_scatter(l_arr, a_arr, b_arr, c_arr,
                           senders.reshape(NW, epw),
                           receivers.reshape(NW, epw),
                           de0, de1, ds0, ds1)

    res = _combine_call(nodebuf, partials)
    dzdt = res[:n, 0:2]
    deg_e = res[:n, 2:4].reshape(n, 2, 1)
    deg_s = res[:n, 4:6].reshape(n, 2, 1)
    return (dzdt, deg_e, deg_s)
